# Initial kernel scaffold; baseline (speedup 1.0000x reference)
import jax
import jax.numpy as jnp
from jax.experimental import pallas as pl


def kernel(position_stack, mass_stack, velocity_stack):
    raise NotImplementedError("write your pallas kernel here")


# speedup vs baseline: 417.5923x; 417.5923x over previous
import functools

import jax
import jax.numpy as jnp
from jax import lax
from jax.experimental import pallas as pl
from jax.experimental.pallas import tpu as pltpu
from jax.experimental.pallas import tpu_sc as plsc

N = 262144
G = 64
NUM_CELLS = G * G * G
INV_CELL = 64.0

NCORES = 2
NSUB = 16
NW = NCORES * NSUB
PPW = N // NW
C = 256
CHUNKS = PPW // C
GPC = C // 16
ROWS_PER_TILE = NUM_CELLS // NSUB

_OFFS = [(i, j, k) for i in (0, 1) for j in (0, 1) for k in (0, 1)]


def _p2g_body(px, py, pz, ms, vx, vy, vz, out,
              pxv, pyv, pzv, msv, vxv, vyv, vzv,
              idx_v, val_m, val_u, val_v, val_w, zbuf,
              pm, pu, pv, pw, sem):
    cid = lax.axis_index("c")
    sid = lax.axis_index("s")
    wid = sid * NCORES + cid

    z16 = jnp.zeros((16,), jnp.float32)
    for i in range(128):
        zbuf[pl.ds(i * 16, 16)] = z16
    for plane in (pm, pu, pv, pw):
        for i in range(ROWS_PER_TILE // 2048):
            pltpu.sync_copy(zbuf, plane.at[pl.ds(sid * ROWS_PER_TILE + i * 2048, 2048)])
    plsc.subcore_barrier()

    def chunk(ci, carry):
        base = wid * PPW + ci * C
        pltpu.sync_copy(px.at[pl.ds(base, C)], pxv)
        pltpu.sync_copy(py.at[pl.ds(base, C)], pyv)
        pltpu.sync_copy(pz.at[pl.ds(base, C)], pzv)
        pltpu.sync_copy(ms.at[pl.ds(base, C)], msv)
        pltpu.sync_copy(vx.at[pl.ds(base, C)], vxv)
        pltpu.sync_copy(vy.at[pl.ds(base, C)], vyv)
        pltpu.sync_copy(vz.at[pl.ds(base, C)], vzv)
        copies = []
        for g in range(GPC):
            s = pl.ds(g * 16, 16)
            relx = pxv[s] * INV_CELL
            rely = pyv[s] * INV_CELL
            relz = pzv[s] * INV_CELL
            bx = relx.astype(jnp.int32)
            by = rely.astype(jnp.int32)
            bz = relz.astype(jnp.int32)
            fx = relx - bx.astype(jnp.float32)
            fy = rely - by.astype(jnp.float32)
            fz = relz - bz.astype(jnp.float32)
            wx = (1.0 - fx, fx)
            wy = (1.0 - fy, fy)
            wz = (1.0 - fz, fz)
            hx = (bx << 12, ((bx + 1) & 63) << 12)
            hy = (by << 6, ((by + 1) & 63) << 6)
            hz = (bz, (bz + 1) & 63)
            m16 = msv[s]
            u16 = vxv[s]
            v16 = vyv[s]
            w16 = vzv[s]
            for o, (i, j, k) in enumerate(_OFFS):
                c = pl.ds(o * 16, 16)
                idx_v[g, c] = hx[i] + hy[j] + hz[k]
                w = wx[i] * wy[j] * wz[k]
                val_m[g, c] = w * m16
                val_u[g, c] = w * u16
                val_v[g, c] = w * v16
                val_w[g, c] = w * w16
            copies.extend([
                pltpu.async_copy(val_m.at[g], pm.at[idx_v.at[g]], sem, add=True),
                pltpu.async_copy(val_u.at[g], pu.at[idx_v.at[g]], sem, add=True),
                pltpu.async_copy(val_v.at[g], pv.at[idx_v.at[g]], sem, add=True),
                pltpu.async_copy(val_w.at[g], pw.at[idx_v.at[g]], sem, add=True),
            ])
        for cp in copies:
            cp.wait()
        return carry

    lax.fori_loop(0, CHUNKS, chunk, 0)

    plsc.subcore_barrier()
    r = pl.ds(sid * ROWS_PER_TILE, ROWS_PER_TILE)
    pltpu.sync_copy(pm.at[r], out.at[cid, 0, r])
    pltpu.sync_copy(pu.at[r], out.at[cid, 1, r])
    pltpu.sync_copy(pv.at[r], out.at[cid, 2, r])
    pltpu.sync_copy(pw.at[r], out.at[cid, 3, r])


_p2g = functools.partial(
    pl.kernel,
    out_type=jax.ShapeDtypeStruct((NCORES, 4, NUM_CELLS), jnp.float32),
    mesh=plsc.VectorSubcoreMesh(core_axis_name="c", subcore_axis_name="s"),
    scratch_types=[
        pltpu.VMEM((C,), jnp.float32),
        pltpu.VMEM((C,), jnp.float32),
        pltpu.VMEM((C,), jnp.float32),
        pltpu.VMEM((C,), jnp.float32),
        pltpu.VMEM((C,), jnp.float32),
        pltpu.VMEM((C,), jnp.float32),
        pltpu.VMEM((C,), jnp.float32),
        pltpu.VMEM((GPC, 128), jnp.int32),
        pltpu.VMEM((GPC, 128), jnp.float32),
        pltpu.VMEM((GPC, 128), jnp.float32),
        pltpu.VMEM((GPC, 128), jnp.float32),
        pltpu.VMEM((GPC, 128), jnp.float32),
        pltpu.VMEM((2048,), jnp.float32),
        pltpu.VMEM_SHARED((NUM_CELLS,), jnp.float32),
        pltpu.VMEM_SHARED((NUM_CELLS,), jnp.float32),
        pltpu.VMEM_SHARED((NUM_CELLS,), jnp.float32),
        pltpu.VMEM_SHARED((NUM_CELLS,), jnp.float32),
        pltpu.SemaphoreType.DMA,
    ],
)(_p2g_body)


def _add_body(a_ref, b_ref, o_ref):
    o_ref[...] = a_ref[...] + b_ref[...]


_combine = pl.pallas_call(
    _add_body,
    out_shape=jax.ShapeDtypeStruct((1024, 1024), jnp.float32),
    grid=(8,),
    in_specs=[pl.BlockSpec((128, 1024), lambda i: (i, 0)),
              pl.BlockSpec((128, 1024), lambda i: (i, 0))],
    out_specs=pl.BlockSpec((128, 1024), lambda i: (i, 0)),
)


def kernel(position_stack, mass_stack, velocity_stack):
    pt = position_stack.T
    vt = velocity_stack.T
    partials = _p2g(pt[0], pt[1], pt[2], mass_stack, vt[0], vt[1], vt[2])
    a = partials[0].reshape(1024, 1024)
    b = partials[1].reshape(1024, 1024)
    planes = _combine(a, b).reshape(4, NUM_CELLS)
    return planes.T


# speedup vs baseline: 707.2970x; 1.6938x over previous
import functools

import jax
import jax.numpy as jnp
from jax import lax
from jax.experimental import pallas as pl
from jax.experimental.pallas import tpu as pltpu
from jax.experimental.pallas import tpu_sc as plsc

N = 262144
G = 64
NUM_CELLS = G * G * G
INV_CELL = 64.0

NCORES = 2
NSUB = 16
NW = NCORES * NSUB
PPW = N // NW
C = 256
CHUNKS = PPW // C
GPC = C // 16
M = 8 * C
ROWS_PER_TILE = NUM_CELLS // NSUB

_OFFS = [(i, j, k) for i in (0, 1) for j in (0, 1) for k in (0, 1)]


def _p2g_body(px, py, pz, ms, vx, vy, vz, out,
              st0, st1, ib0, ib1, zbuf,
              pm, pu, pv, pw,
              ssem0, ssem1, vsem0, vsem1):
    cid = lax.axis_index("c")
    sid = lax.axis_index("s")
    wid = sid * NCORES + cid

    stages = (st0, st1)
    ibufs = (ib0, ib1)
    ssems = (ssem0, ssem1)
    vsems = (vsem0, vsem1)
    planes = (pm, pu, pv, pw)
    inputs = (px, py, pz, ms, vx, vy, vz)

    def fire_stage(p, ci):
        b = wid * PPW + jnp.minimum(ci, CHUNKS - 1) * C
        for src, dst in zip(inputs, stages[p]):
            pltpu.async_copy(src.at[pl.ds(b, C)], dst, ssems[p])

    def wait_stage(p):
        for src, dst in zip(inputs, stages[p]):
            pltpu.make_async_copy(src.at[pl.ds(0, C)], dst, ssems[p]).wait()

    def compute_fire(p, ci):
        pxv, pyv, pzv, msv, vxv, vyv, vzv = stages[p]
        idx_v, val_m, val_u, val_v, val_w = ibufs[p]
        for g in range(GPC):
            s = pl.ds(g * 16, 16)
            relx = pxv[s] * INV_CELL
            rely = pyv[s] * INV_CELL
            relz = pzv[s] * INV_CELL
            bx = relx.astype(jnp.int32)
            by = rely.astype(jnp.int32)
            bz = relz.astype(jnp.int32)
            fx = relx - bx.astype(jnp.float32)
            fy = rely - by.astype(jnp.float32)
            fz = relz - bz.astype(jnp.float32)
            wx = (1.0 - fx, fx)
            wy = (1.0 - fy, fy)
            wz = (1.0 - fz, fz)
            hx = (bx << 12, ((bx + 1) & 63) << 12)
            hy = (by << 6, ((by + 1) & 63) << 6)
            hz = (bz, (bz + 1) & 63)
            m16 = msv[s]
            u16 = vxv[s]
            v16 = vyv[s]
            w16 = vzv[s]
            for o, (i, j, k) in enumerate(_OFFS):
                c = pl.ds(g * 128 + o * 16, 16)
                idx_v[c] = hx[i] + hy[j] + hz[k]
                w = wx[i] * wy[j] * wz[k]
                val_m[c] = w * m16
                val_u[c] = w * u16
                val_v[c] = w * v16
                val_w[c] = w * w16
        for val, plane in zip((val_m, val_u, val_v, val_w), planes):
            pltpu.async_copy(val, plane.at[idx_v], vsems[p], add=True)

    def wait_streams(p):
        idx_v, val_m, val_u, val_v, val_w = ibufs[p]
        for val, plane in zip((val_m, val_u, val_v, val_w), planes):
            pltpu.make_async_copy(val, plane.at[idx_v], vsems[p]).wait()

    fire_stage(0, 0)
    fire_stage(1, 1)
    z16 = jnp.zeros((16,), jnp.float32)
    for i in range(128):
        zbuf[pl.ds(i * 16, 16)] = z16
    for plane in planes:
        for i in range(ROWS_PER_TILE // 2048):
            pltpu.sync_copy(zbuf, plane.at[pl.ds(sid * ROWS_PER_TILE + i * 2048, 2048)])
    plsc.subcore_barrier()

    wait_stage(0)
    compute_fire(0, 0)
    fire_stage(0, 2)
    wait_stage(1)
    compute_fire(1, 1)
    fire_stage(1, 3)

    def body(i, carry):
        wait_streams(0)
        wait_stage(0)
        compute_fire(0, 2 + 2 * i)
        fire_stage(0, 4 + 2 * i)
        wait_streams(1)
        wait_stage(1)
        compute_fire(1, 3 + 2 * i)
        fire_stage(1, 5 + 2 * i)
        return carry

    lax.fori_loop(0, (CHUNKS - 2) // 2, body, 0)
    wait_streams(0)
    wait_stage(0)
    wait_streams(1)
    wait_stage(1)

    plsc.subcore_barrier()
    r = pl.ds(sid * ROWS_PER_TILE, ROWS_PER_TILE)
    pltpu.sync_copy(pm.at[r], out.at[cid, 0, r])
    pltpu.sync_copy(pu.at[r], out.at[cid, 1, r])
    pltpu.sync_copy(pv.at[r], out.at[cid, 2, r])
    pltpu.sync_copy(pw.at[r], out.at[cid, 3, r])


def _stage_types():
    return tuple(pltpu.VMEM((C,), jnp.float32) for _ in range(7))


def _ibuf_types():
    return (pltpu.VMEM((M,), jnp.int32),) + tuple(
        pltpu.VMEM((M,), jnp.float32) for _ in range(4))


_p2g = functools.partial(
    pl.kernel,
    out_type=jax.ShapeDtypeStruct((NCORES, 4, NUM_CELLS), jnp.float32),
    mesh=plsc.VectorSubcoreMesh(core_axis_name="c", subcore_axis_name="s"),
    scratch_types=[
        _stage_types(),
        _stage_types(),
        _ibuf_types(),
        _ibuf_types(),
        pltpu.VMEM((2048,), jnp.float32),
        pltpu.VMEM_SHARED((NUM_CELLS,), jnp.float32),
        pltpu.VMEM_SHARED((NUM_CELLS,), jnp.float32),
        pltpu.VMEM_SHARED((NUM_CELLS,), jnp.float32),
        pltpu.VMEM_SHARED((NUM_CELLS,), jnp.float32),
        pltpu.SemaphoreType.DMA,
        pltpu.SemaphoreType.DMA,
        pltpu.SemaphoreType.DMA,
        pltpu.SemaphoreType.DMA,
    ],
)(_p2g_body)


def _add_body(a_ref, b_ref, o_ref):
    o_ref[...] = a_ref[...] + b_ref[...]


_combine = pl.pallas_call(
    _add_body,
    out_shape=jax.ShapeDtypeStruct((1024, 1024), jnp.float32),
    grid=(8,),
    in_specs=[pl.BlockSpec((128, 1024), lambda i: (i, 0)),
              pl.BlockSpec((128, 1024), lambda i: (i, 0))],
    out_specs=pl.BlockSpec((128, 1024), lambda i: (i, 0)),
)


def kernel(position_stack, mass_stack, velocity_stack):
    pt = position_stack.T
    vt = velocity_stack.T
    partials = _p2g(pt[0], pt[1], pt[2], mass_stack, vt[0], vt[1], vt[2])
    a = partials[0].reshape(1024, 1024)
    b = partials[1].reshape(1024, 1024)
    planes = _combine(a, b).reshape(4, NUM_CELLS)
    return planes.T


# speedup vs baseline: 787.5830x; 1.1135x over previous
import functools

import jax
import jax.numpy as jnp
from jax import lax
from jax.experimental import pallas as pl
from jax.experimental.pallas import tpu as pltpu
from jax.experimental.pallas import tpu_sc as plsc

N = 262144
G = 64
NUM_CELLS = G * G * G
INV_CELL = 64.0

NCORES = 2
NSUB = 16
NW = NCORES * NSUB
PPW = N // NW
C = 256
CHUNKS = PPW // C
GPC = C // 16
M = 8 * C
ROWS_PER_TILE = NUM_CELLS // NSUB

_OFFS = [(i, j, k) for i in (0, 1) for j in (0, 1) for k in (0, 1)]


def _p2g_body(px, py, pz, ms, vx, vy, vz, out,
              st0, st1, ib0, ib1, zbuf,
              pm, pu, pv, pw,
              ssem0, ssem1, vsem0, vsem1):
    cid = lax.axis_index("c")
    sid = lax.axis_index("s")
    wid = sid * NCORES + cid

    stages = (st0, st1)
    ibufs = (ib0, ib1)
    ssems = (ssem0, ssem1)
    vsems = (vsem0, vsem1)
    planes = (pm, pu, pv, pw)
    inputs = (px, py, pz, ms, vx, vy, vz)

    def fire_stage(p, ci):
        b = wid * PPW + jnp.minimum(ci, CHUNKS - 1) * C
        for src, dst in zip(inputs, stages[p]):
            pltpu.async_copy(src.at[pl.ds(b, C)], dst, ssems[p])

    def wait_stage(p):
        for src, dst in zip(inputs, stages[p]):
            pltpu.make_async_copy(src.at[pl.ds(0, C)], dst, ssems[p]).wait()

    def compute_fire(p, ci):
        pxv, pyv, pzv, msv, vxv, vyv, vzv = stages[p]
        idx_v, val_m, val_u, val_v, val_w = ibufs[p]
        for g in range(GPC):
            s = pl.ds(g * 16, 16)
            relx = pxv[s] * INV_CELL
            rely = pyv[s] * INV_CELL
            relz = pzv[s] * INV_CELL
            bx = relx.astype(jnp.int32)
            by = rely.astype(jnp.int32)
            bz = relz.astype(jnp.int32)
            fx = relx - bx.astype(jnp.float32)
            fy = rely - by.astype(jnp.float32)
            fz = relz - bz.astype(jnp.float32)
            wx = (1.0 - fx, fx)
            wy = (1.0 - fy, fy)
            wz = (1.0 - fz, fz)
            hx = (bx << 12, ((bx + 1) & 63) << 12)
            hy = (by << 6, ((by + 1) & 63) << 6)
            hz = (bz, (bz + 1) & 63)
            m16 = msv[s]
            u16 = vxv[s]
            v16 = vyv[s]
            w16 = vzv[s]
            for o, (i, j, k) in enumerate(_OFFS):
                c = pl.ds(g * 128 + o * 16, 16)
                idx_v[c] = hx[i] + hy[j] + hz[k]
                w = wx[i] * wy[j] * wz[k]
                val_m[c] = w * m16
                val_u[c] = w * u16
                val_v[c] = w * v16
                val_w[c] = w * w16
        for val, plane in zip((val_m, val_u, val_v, val_w), planes):
            pltpu.async_copy(val, plane.at[idx_v], vsems[p], add=True)

    def wait_streams(p):
        idx_v, val_m, val_u, val_v, val_w = ibufs[p]
        for val, plane in zip((val_m, val_u, val_v, val_w), planes):
            pltpu.make_async_copy(val, plane.at[idx_v], vsems[p]).wait()

    fire_stage(0, 0)
    fire_stage(1, 1)
    z16 = jnp.zeros((16,), jnp.float32)
    for i in range(128):
        zbuf[pl.ds(i * 16, 16)] = z16
    for plane in planes:
        for i in range(ROWS_PER_TILE // 2048):
            pltpu.sync_copy(zbuf, plane.at[pl.ds(sid * ROWS_PER_TILE + i * 2048, 2048)])
    plsc.subcore_barrier()

    wait_stage(0)
    compute_fire(0, 0)
    fire_stage(0, 2)
    wait_stage(1)
    compute_fire(1, 1)
    fire_stage(1, 3)

    def body(i, carry):
        wait_streams(0)
        wait_stage(0)
        compute_fire(0, 2 + 2 * i)
        fire_stage(0, 4 + 2 * i)
        wait_streams(1)
        wait_stage(1)
        compute_fire(1, 3 + 2 * i)
        fire_stage(1, 5 + 2 * i)
        return carry

    lax.fori_loop(0, (CHUNKS - 2) // 2, body, 0)
    wait_streams(0)
    wait_stage(0)
    wait_streams(1)
    wait_stage(1)

    plsc.subcore_barrier()
    r = pl.ds(sid * ROWS_PER_TILE, ROWS_PER_TILE)
    for ch, plane in enumerate(planes):
        pltpu.sync_copy(
            plane.at[r],
            out.at[cid, pl.ds(ch * NUM_CELLS + sid * ROWS_PER_TILE, ROWS_PER_TILE)])


def _stage_types():
    return tuple(pltpu.VMEM((C,), jnp.float32) for _ in range(7))


def _ibuf_types():
    return (pltpu.VMEM((M,), jnp.int32),) + tuple(
        pltpu.VMEM((M,), jnp.float32) for _ in range(4))


_p2g = functools.partial(
    pl.kernel,
    out_type=jax.ShapeDtypeStruct((NCORES, 4 * NUM_CELLS), jnp.float32),
    mesh=plsc.VectorSubcoreMesh(core_axis_name="c", subcore_axis_name="s"),
    scratch_types=[
        _stage_types(),
        _stage_types(),
        _ibuf_types(),
        _ibuf_types(),
        pltpu.VMEM((2048,), jnp.float32),
        pltpu.VMEM_SHARED((NUM_CELLS,), jnp.float32),
        pltpu.VMEM_SHARED((NUM_CELLS,), jnp.float32),
        pltpu.VMEM_SHARED((NUM_CELLS,), jnp.float32),
        pltpu.VMEM_SHARED((NUM_CELLS,), jnp.float32),
        pltpu.SemaphoreType.DMA,
        pltpu.SemaphoreType.DMA,
        pltpu.SemaphoreType.DMA,
        pltpu.SemaphoreType.DMA,
    ],
)(_p2g_body)


CPT = 4 * NUM_CELLS // NW


def _comb_body(parts, out, b0, b1, sem):
    cid = lax.axis_index("c")
    sid = lax.axis_index("s")
    wid = sid * NCORES + cid
    off = wid * CPT
    cp0 = pltpu.async_copy(parts.at[0, pl.ds(off, CPT)], b0, sem)
    cp1 = pltpu.async_copy(parts.at[1, pl.ds(off, CPT)], b1, sem)
    cp0.wait()
    cp1.wait()

    def add_block(i, carry):
        base = pl.multiple_of(i * 256, 256)
        for j in range(16):
            s = pl.ds(base + j * 16, 16)
            b0[s] = b0[s] + b1[s]
        return carry

    lax.fori_loop(0, CPT // 256, add_block, 0)
    pltpu.sync_copy(b0, out.at[pl.ds(off, CPT)])


_combine = functools.partial(
    pl.kernel,
    out_type=jax.ShapeDtypeStruct((4 * NUM_CELLS,), jnp.float32),
    mesh=plsc.VectorSubcoreMesh(core_axis_name="c", subcore_axis_name="s"),
    scratch_types=[
        pltpu.VMEM((CPT,), jnp.float32),
        pltpu.VMEM((CPT,), jnp.float32),
        pltpu.SemaphoreType.DMA,
    ],
)(_comb_body)


def kernel(position_stack, mass_stack, velocity_stack):
    pt = position_stack.T
    vt = velocity_stack.T
    partials = _p2g(pt[0], pt[1], pt[2], mass_stack, vt[0], vt[1], vt[2])
    return _combine(partials).reshape(4, NUM_CELLS).T


# speedup vs baseline: 807.2726x; 1.0250x over previous
import functools

import jax
import jax.numpy as jnp
from jax import lax
from jax.experimental import pallas as pl
from jax.experimental.pallas import tpu as pltpu
from jax.experimental.pallas import tpu_sc as plsc

N = 262144
G = 64
NUM_CELLS = G * G * G
INV_CELL = 64.0

NCORES = 2
NSUB = 16
NW = NCORES * NSUB
PPW = N // NW
C = 512
CHUNKS = PPW // C
GPC = C // 16
M = 8 * C
ROWS_PER_TILE = NUM_CELLS // NSUB

_OFFS = [(i, j, k) for i in (0, 1) for j in (0, 1) for k in (0, 1)]


def _p2g_body(px, py, pz, ms, vx, vy, vz, out,
              st0, st1, ib0, ib1, zbuf,
              pm, pu, pv, pw,
              ssem0, ssem1, vsem0, vsem1):
    cid = lax.axis_index("c")
    sid = lax.axis_index("s")
    wid = sid * NCORES + cid

    stages = (st0, st1)
    ibufs = (ib0, ib1)
    ssems = (ssem0, ssem1)
    vsems = (vsem0, vsem1)
    planes = (pm, pu, pv, pw)
    inputs = (px, py, pz, ms, vx, vy, vz)

    def fire_stage(p, ci):
        b = wid * PPW + jnp.minimum(ci, CHUNKS - 1) * C
        for src, dst in zip(inputs, stages[p]):
            pltpu.async_copy(src.at[pl.ds(b, C)], dst, ssems[p])

    def wait_stage(p):
        for src, dst in zip(inputs, stages[p]):
            pltpu.make_async_copy(src.at[pl.ds(0, C)], dst, ssems[p]).wait()

    def compute_fire(p, ci):
        pxv, pyv, pzv, msv, vxv, vyv, vzv = stages[p]
        idx_v, val_m, val_u, val_v, val_w = ibufs[p]

        def group(g, carry):
            g16 = pl.multiple_of(g * 16, 16)
            g128 = pl.multiple_of(g * 128, 128)
            s = pl.ds(g16, 16)
            relx = pxv[s] * INV_CELL
            rely = pyv[s] * INV_CELL
            relz = pzv[s] * INV_CELL
            bx = relx.astype(jnp.int32)
            by = rely.astype(jnp.int32)
            bz = relz.astype(jnp.int32)
            fx = relx - bx.astype(jnp.float32)
            fy = rely - by.astype(jnp.float32)
            fz = relz - bz.astype(jnp.float32)
            wx = (1.0 - fx, fx)
            wy = (1.0 - fy, fy)
            wz = (1.0 - fz, fz)
            hx = (bx << 12, ((bx + 1) & 63) << 12)
            hy = (by << 6, ((by + 1) & 63) << 6)
            hz = (bz, (bz + 1) & 63)
            m16 = msv[s]
            u16 = vxv[s]
            v16 = vyv[s]
            w16 = vzv[s]
            for o, (i, j, k) in enumerate(_OFFS):
                c = pl.ds(g128 + o * 16, 16)
                idx_v[c] = hx[i] + hy[j] + hz[k]
                w = wx[i] * wy[j] * wz[k]
                val_m[c] = w * m16
                val_u[c] = w * u16
                val_v[c] = w * v16
                val_w[c] = w * w16
            return carry

        lax.fori_loop(0, GPC, group, 0)
        for val, plane in zip((val_m, val_u, val_v, val_w), planes):
            pltpu.async_copy(val, plane.at[idx_v], vsems[p], add=True)

    def wait_streams(p):
        idx_v, val_m, val_u, val_v, val_w = ibufs[p]
        for val, plane in zip((val_m, val_u, val_v, val_w), planes):
            pltpu.make_async_copy(val, plane.at[idx_v], vsems[p]).wait()

    fire_stage(0, 0)
    fire_stage(1, 1)
    z16 = jnp.zeros((16,), jnp.float32)
    for i in range(128):
        zbuf[pl.ds(i * 16, 16)] = z16
    for plane in planes:
        for i in range(ROWS_PER_TILE // 2048):
            pltpu.sync_copy(zbuf, plane.at[pl.ds(sid * ROWS_PER_TILE + i * 2048, 2048)])
    plsc.subcore_barrier()

    wait_stage(0)
    compute_fire(0, 0)
    fire_stage(0, 2)
    wait_stage(1)
    compute_fire(1, 1)
    fire_stage(1, 3)

    def body(i, carry):
        wait_streams(0)
        wait_stage(0)
        compute_fire(0, 2 + 2 * i)
        fire_stage(0, 4 + 2 * i)
        wait_streams(1)
        wait_stage(1)
        compute_fire(1, 3 + 2 * i)
        fire_stage(1, 5 + 2 * i)
        return carry

    lax.fori_loop(0, (CHUNKS - 2) // 2, body, 0)
    wait_streams(0)
    wait_stage(0)
    wait_streams(1)
    wait_stage(1)

    plsc.subcore_barrier()
    r = pl.ds(sid * ROWS_PER_TILE, ROWS_PER_TILE)
    for ch, plane in enumerate(planes):
        pltpu.sync_copy(
            plane.at[r],
            out.at[cid, pl.ds(ch * NUM_CELLS + sid * ROWS_PER_TILE, ROWS_PER_TILE)])


def _stage_types():
    return tuple(pltpu.VMEM((C,), jnp.float32) for _ in range(7))


def _ibuf_types():
    return (pltpu.VMEM((M,), jnp.int32),) + tuple(
        pltpu.VMEM((M,), jnp.float32) for _ in range(4))


_p2g = functools.partial(
    pl.kernel,
    out_type=jax.ShapeDtypeStruct((NCORES, 4 * NUM_CELLS), jnp.float32),
    mesh=plsc.VectorSubcoreMesh(core_axis_name="c", subcore_axis_name="s"),
    scratch_types=[
        _stage_types(),
        _stage_types(),
        _ibuf_types(),
        _ibuf_types(),
        pltpu.VMEM((2048,), jnp.float32),
        pltpu.VMEM_SHARED((NUM_CELLS,), jnp.float32),
        pltpu.VMEM_SHARED((NUM_CELLS,), jnp.float32),
        pltpu.VMEM_SHARED((NUM_CELLS,), jnp.float32),
        pltpu.VMEM_SHARED((NUM_CELLS,), jnp.float32),
        pltpu.SemaphoreType.DMA,
        pltpu.SemaphoreType.DMA,
        pltpu.SemaphoreType.DMA,
        pltpu.SemaphoreType.DMA,
    ],
)(_p2g_body)


CPT = 4 * NUM_CELLS // NW


def _comb_body(parts, out, b0, b1, sem):
    cid = lax.axis_index("c")
    sid = lax.axis_index("s")
    wid = sid * NCORES + cid
    off = wid * CPT
    cp0 = pltpu.async_copy(parts.at[0, pl.ds(off, CPT)], b0, sem)
    cp1 = pltpu.async_copy(parts.at[1, pl.ds(off, CPT)], b1, sem)
    cp0.wait()
    cp1.wait()

    def add_block(i, carry):
        base = pl.multiple_of(i * 256, 256)
        for j in range(16):
            s = pl.ds(base + j * 16, 16)
            b0[s] = b0[s] + b1[s]
        return carry

    lax.fori_loop(0, CPT // 256, add_block, 0)
    pltpu.sync_copy(b0, out.at[pl.ds(off, CPT)])


_combine = functools.partial(
    pl.kernel,
    out_type=jax.ShapeDtypeStruct((4 * NUM_CELLS,), jnp.float32),
    mesh=plsc.VectorSubcoreMesh(core_axis_name="c", subcore_axis_name="s"),
    scratch_types=[
        pltpu.VMEM((CPT,), jnp.float32),
        pltpu.VMEM((CPT,), jnp.float32),
        pltpu.SemaphoreType.DMA,
    ],
)(_comb_body)


def kernel(position_stack, mass_stack, velocity_stack):
    pt = position_stack.T
    vt = velocity_stack.T
    partials = _p2g(pt[0], pt[1], pt[2], mass_stack, vt[0], vt[1], vt[2])
    return _combine(partials).reshape(4, NUM_CELLS).T


# speedup vs baseline: 808.4805x; 1.0015x over previous
import functools

import jax
import jax.numpy as jnp
from jax import lax
from jax.experimental import pallas as pl
from jax.experimental.pallas import tpu as pltpu
from jax.experimental.pallas import tpu_sc as plsc

N = 262144
G = 64
NUM_CELLS = G * G * G
INV_CELL = 64.0

NCORES = 2
NSUB = 16
NW = NCORES * NSUB
PPW = N // NW
C = 512
CHUNKS = PPW // C
GPC = C // 16
M = 8 * C
ROWS_PER_TILE = NUM_CELLS // NSUB

_OFFS = [(i, j, k) for i in (0, 1) for j in (0, 1) for k in (0, 1)]


def _p2g_body(px, py, pz, ms, vx, vy, vz, out,
              st0, st1, ib0, ib1, zbuf,
              pm, pu, pv, pw,
              ssem0, ssem1, vsem0, vsem1):
    cid = lax.axis_index("c")
    sid = lax.axis_index("s")
    wid = sid * NCORES + cid

    stages = (st0, st1)
    ibufs = (ib0, ib1)
    ssems = (ssem0, ssem1)
    vsems = (vsem0, vsem1)
    planes = (pm, pu, pv, pw)
    inputs = (px, py, pz, ms, vx, vy, vz)

    def fire_stage(p, ci):
        b = wid * PPW + jnp.minimum(ci, CHUNKS - 1) * C
        for src, dst in zip(inputs, stages[p]):
            pltpu.async_copy(src.at[pl.ds(b, C)], dst, ssems[p])

    def wait_stage(p):
        for src, dst in zip(inputs, stages[p]):
            pltpu.make_async_copy(src.at[pl.ds(0, C)], dst, ssems[p]).wait()

    def compute_fire(p, ci):
        pxv, pyv, pzv, msv, vxv, vyv, vzv = stages[p]
        idx_v, val_m, val_u, val_v, val_w = ibufs[p]

        def group(g, carry):
            g16 = pl.multiple_of(g * 16, 16)
            g128 = pl.multiple_of(g * 128, 128)
            s = pl.ds(g16, 16)
            relx = pxv[s] * INV_CELL
            rely = pyv[s] * INV_CELL
            relz = pzv[s] * INV_CELL
            bx = relx.astype(jnp.int32)
            by = rely.astype(jnp.int32)
            bz = relz.astype(jnp.int32)
            fx = relx - bx.astype(jnp.float32)
            fy = rely - by.astype(jnp.float32)
            fz = relz - bz.astype(jnp.float32)
            wx = (1.0 - fx, fx)
            wy = (1.0 - fy, fy)
            wz = (1.0 - fz, fz)
            hx = (bx << 12, ((bx + 1) & 63) << 12)
            hy = (by << 6, ((by + 1) & 63) << 6)
            hz = (bz, (bz + 1) & 63)
            m16 = msv[s]
            u16 = vxv[s]
            v16 = vyv[s]
            w16 = vzv[s]
            for o, (i, j, k) in enumerate(_OFFS):
                c = pl.ds(g128 + o * 16, 16)
                idx_v[c] = hx[i] + hy[j] + hz[k]
                w = wx[i] * wy[j] * wz[k]
                val_m[c] = w * m16
                val_u[c] = w * u16
                val_v[c] = w * v16
                val_w[c] = w * w16
            return carry

        lax.fori_loop(0, GPC, group, 0)
        for val, plane in zip((val_m, val_u, val_v, val_w), planes):
            pltpu.async_copy(val, plane.at[idx_v], vsems[p], add=True)

    def wait_streams(p):
        idx_v, val_m, val_u, val_v, val_w = ibufs[p]
        for val, plane in zip((val_m, val_u, val_v, val_w), planes):
            pltpu.make_async_copy(val, plane.at[idx_v], vsems[p]).wait()

    fire_stage(0, 0)
    fire_stage(1, 1)
    z16 = jnp.zeros((16,), jnp.float32)
    for i in range(128):
        zbuf[pl.ds(i * 16, 16)] = z16
    for plane in planes:
        for i in range(ROWS_PER_TILE // 2048):
            pltpu.sync_copy(zbuf, plane.at[pl.ds(sid * ROWS_PER_TILE + i * 2048, 2048)])
    plsc.subcore_barrier()

    wait_stage(0)
    compute_fire(0, 0)
    fire_stage(0, 2)
    wait_stage(1)
    compute_fire(1, 1)
    fire_stage(1, 3)

    def body(i, carry):
        wait_streams(0)
        wait_stage(0)
        compute_fire(0, 2 + 2 * i)
        fire_stage(0, 4 + 2 * i)
        wait_streams(1)
        wait_stage(1)
        compute_fire(1, 3 + 2 * i)
        fire_stage(1, 5 + 2 * i)
        return carry

    lax.fori_loop(0, (CHUNKS - 2) // 2, body, 0)
    wait_streams(0)
    wait_stage(0)
    wait_streams(1)
    wait_stage(1)

    plsc.subcore_barrier()
    r = pl.ds(sid * ROWS_PER_TILE, ROWS_PER_TILE)
    for ch, plane in enumerate(planes):
        pltpu.sync_copy(
            plane.at[r],
            out.at[cid, pl.ds(ch * NUM_CELLS + sid * ROWS_PER_TILE, ROWS_PER_TILE)])


def _stage_types():
    return tuple(pltpu.VMEM((C,), jnp.float32) for _ in range(7))


def _ibuf_types():
    return (pltpu.VMEM((M,), jnp.int32),) + tuple(
        pltpu.VMEM((M,), jnp.float32) for _ in range(4))


_p2g = functools.partial(
    pl.kernel,
    out_type=jax.ShapeDtypeStruct((NCORES, 4 * NUM_CELLS), jnp.float32),
    mesh=plsc.VectorSubcoreMesh(core_axis_name="c", subcore_axis_name="s"),
    scratch_types=[
        _stage_types(),
        _stage_types(),
        _ibuf_types(),
        _ibuf_types(),
        pltpu.VMEM((2048,), jnp.float32),
        pltpu.VMEM_SHARED((NUM_CELLS,), jnp.float32),
        pltpu.VMEM_SHARED((NUM_CELLS,), jnp.float32),
        pltpu.VMEM_SHARED((NUM_CELLS,), jnp.float32),
        pltpu.VMEM_SHARED((NUM_CELLS,), jnp.float32),
        pltpu.SemaphoreType.DMA,
        pltpu.SemaphoreType.DMA,
        pltpu.SemaphoreType.DMA,
        pltpu.SemaphoreType.DMA,
    ],
)(_p2g_body)


CPT = 4 * NUM_CELLS // NW
CQ = 4
CB = CPT // CQ


def _comb_body(parts, out, ca0, cb0, ca1, cb1, isem0, isem1, osem0, osem1):
    cid = lax.axis_index("c")
    sid = lax.axis_index("s")
    wid = sid * NCORES + cid
    off = wid * CPT
    abufs = (ca0, ca1)
    bbufs = (cb0, cb1)
    isems = (isem0, isem1)
    osems = (osem0, osem1)

    def fire_in(p, c):
        o = off + c * CB
        pltpu.async_copy(parts.at[0, pl.ds(o, CB)], abufs[p], isems[p])
        pltpu.async_copy(parts.at[1, pl.ds(o, CB)], bbufs[p], isems[p])

    def wait_in(p):
        pltpu.make_async_copy(parts.at[0, pl.ds(0, CB)], abufs[p], isems[p]).wait()
        pltpu.make_async_copy(parts.at[1, pl.ds(0, CB)], bbufs[p], isems[p]).wait()

    fire_in(0, 0)
    fire_in(1, 1)
    for c in range(CQ):
        p = c % 2
        wait_in(p)
        a, b = abufs[p], bbufs[p]

        def add_block(i, carry):
            base = pl.multiple_of(i * 256, 256)
            for j in range(16):
                s = pl.ds(base + j * 16, 16)
                a[s] = a[s] + b[s]
            return carry

        lax.fori_loop(0, CB // 256, add_block, 0)
        if c >= 2:
            pltpu.make_async_copy(abufs[p], out.at[pl.ds(0, CB)], osems[p]).wait()
        pltpu.async_copy(a, out.at[pl.ds(off + c * CB, CB)], osems[p])
        if c + 2 < CQ:
            fire_in(p, c + 2)
    for p in range(2):
        pltpu.make_async_copy(abufs[p], out.at[pl.ds(0, CB)], osems[p]).wait()


_combine = functools.partial(
    pl.kernel,
    out_type=jax.ShapeDtypeStruct((4 * NUM_CELLS,), jnp.float32),
    mesh=plsc.VectorSubcoreMesh(core_axis_name="c", subcore_axis_name="s"),
    scratch_types=[
        pltpu.VMEM((CB,), jnp.float32),
        pltpu.VMEM((CB,), jnp.float32),
        pltpu.VMEM((CB,), jnp.float32),
        pltpu.VMEM((CB,), jnp.float32),
        pltpu.SemaphoreType.DMA,
        pltpu.SemaphoreType.DMA,
        pltpu.SemaphoreType.DMA,
        pltpu.SemaphoreType.DMA,
    ],
)(_comb_body)


def kernel(position_stack, mass_stack, velocity_stack):
    pt = position_stack.T
    vt = velocity_stack.T
    partials = _p2g(pt[0], pt[1], pt[2], mass_stack, vt[0], vt[1], vt[2])
    return _combine(partials).reshape(4, NUM_CELLS).T
